# Initial kernel scaffold; baseline (speedup 1.0000x reference)
import jax
import jax.numpy as jnp
from jax.experimental import pallas as pl


def kernel(x, hidden_h, hidden_c, table):
    raise NotImplementedError("write your pallas kernel here")


# speedup vs baseline: 7.2995x; 7.2995x over previous
import functools

import jax
import jax.numpy as jnp
from jax import lax
from jax.experimental import pallas as pl
from jax.experimental.pallas import tpu as pltpu
from jax.experimental.pallas import tpu_sc as plsc

BATCH = 1024
SEQ = 200
EMBED = 128
N = BATCH * SEQ
NW = 32
PER_W = N // NW
CHUNK = 128
NCH = PER_W // CHUNK
K = 5
NG = NCH // K


def _emb(idx_hbm, table_hbm, out_hbm, idx_v, rows_v, gsem, wsem):
    nc = 2
    wid = lax.axis_index("s") * nc + lax.axis_index("c")
    base = wid * PER_W
    pltpu.sync_copy(idx_hbm.at[wid], idx_v)

    def body(g, carry):
        j0 = g * K
        gh = []
        for b in range(K):
            gh.append(pltpu.async_copy(
                table_hbm.at[idx_v.at[j0 + b]], rows_v.at[b], gsem))
        for b in range(K):
            gh[b].wait()
        wh = []
        for b in range(K):
            wh.append(pltpu.async_copy(
                rows_v.at[b],
                out_hbm.at[pl.ds(base + (j0 + b) * CHUNK, CHUNK)], wsem))
        for b in range(K):
            wh[b].wait()
        return carry

    lax.fori_loop(0, NG, body, 0)


@jax.jit
def _lookup(idx, table):
    mesh = plsc.VectorSubcoreMesh(core_axis_name="c", subcore_axis_name="s")
    return pl.kernel(
        _emb,
        out_type=jax.ShapeDtypeStruct((N, EMBED), jnp.float32),
        mesh=mesh,
        scratch_types=[
            pltpu.VMEM((NCH, CHUNK), jnp.int32),
            pltpu.VMEM((K, CHUNK, EMBED), jnp.float32),
            pltpu.SemaphoreType.DMA,
            pltpu.SemaphoreType.DMA,
        ],
    )(idx, table)


def kernel(x, hidden_h, hidden_c, table):
    idx = x.reshape(NW, NCH, CHUNK)
    embeds = _lookup(idx, table).reshape(BATCH, SEQ, EMBED)
    return (embeds, hidden_h, hidden_c)


# speedup vs baseline: 7.4711x; 1.0235x over previous
import functools

import jax
import jax.numpy as jnp
from jax import lax
from jax.experimental import pallas as pl
from jax.experimental.pallas import tpu as pltpu
from jax.experimental.pallas import tpu_sc as plsc

BATCH = 1024
SEQ = 200
EMBED = 128
N = BATCH * SEQ
NW = 32
PER_W = N // NW
CHUNK = 128
NCH = PER_W // CHUNK
K = 5
NG = NCH // K


def _emb(idx_hbm, table_hbm, out_hbm, idx_v, rows_v, gsem, wsem):
    nc = 2
    wid = lax.axis_index("s") * nc + lax.axis_index("c")
    base = wid * PER_W
    pltpu.sync_copy(idx_hbm.at[wid], idx_v)

    def gather(j, b):
        return pltpu.async_copy(
            table_hbm.at[idx_v.at[j]], rows_v.at[b], gsem.at[b])

    def write(j, b):
        return pltpu.async_copy(
            rows_v.at[b], out_hbm.at[pl.ds(base + j * CHUNK, CHUNK)],
            wsem.at[b])

    for b in range(K):
        gather(b, b)

    def body(g, carry):
        for b in range(K):
            j = g * K + b
            pltpu.make_async_copy(
                table_hbm.at[idx_v.at[j]], rows_v.at[b], gsem.at[b]).wait()
            write(j, b)
        for b in range(K):
            nj = (g + 1) * K + b
            @pl.when(nj < NCH)
            def _():
                pltpu.make_async_copy(
                    rows_v.at[b],
                    out_hbm.at[pl.ds(base, CHUNK)], wsem.at[b]).wait()
                gather(nj, b)
        return carry

    lax.fori_loop(0, NG, body, 0)
    for b in range(K):
        pltpu.make_async_copy(
            rows_v.at[b], out_hbm.at[pl.ds(base, CHUNK)], wsem.at[b]).wait()


@jax.jit
def _lookup(idx, table):
    mesh = plsc.VectorSubcoreMesh(core_axis_name="c", subcore_axis_name="s")
    return pl.kernel(
        _emb,
        out_type=jax.ShapeDtypeStruct((N, EMBED), jnp.float32),
        mesh=mesh,
        scratch_types=[
            pltpu.VMEM((NCH, CHUNK), jnp.int32),
            pltpu.VMEM((K, CHUNK, EMBED), jnp.float32),
            pltpu.SemaphoreType.DMA((K,)),
            pltpu.SemaphoreType.DMA((K,)),
        ],
    )(idx, table)


def kernel(x, hidden_h, hidden_c, table):
    idx = x.reshape(NW, NCH, CHUNK)
    embeds = _lookup(idx, table).reshape(BATCH, SEQ, EMBED)
    return (embeds, hidden_h, hidden_c)


# speedup vs baseline: 7.5377x; 1.0089x over previous
import functools

import jax
import jax.numpy as jnp
from jax import lax
from jax.experimental import pallas as pl
from jax.experimental.pallas import tpu as pltpu
from jax.experimental.pallas import tpu_sc as plsc

BATCH = 1024
SEQ = 200
EMBED = 128
N = BATCH * SEQ
NW = 32
PER_W = N // NW
CHUNK = 64
NCH = PER_W // CHUNK
K = 10
NG = NCH // K


def _emb(idx_hbm, table_hbm, out_hbm, idx_v, rows_v, gsem, wsem):
    nc = 2
    wid = lax.axis_index("s") * nc + lax.axis_index("c")
    base = wid * PER_W
    pltpu.sync_copy(idx_hbm.at[wid], idx_v)

    def gather(j, b):
        return pltpu.async_copy(
            table_hbm.at[idx_v.at[j]], rows_v.at[b], gsem.at[b])

    def write(j, b):
        return pltpu.async_copy(
            rows_v.at[b], out_hbm.at[pl.ds(base + j * CHUNK, CHUNK)],
            wsem.at[b])

    for b in range(K):
        gather(b, b)

    def body(g, carry):
        for b in range(K):
            j = g * K + b
            pltpu.make_async_copy(
                table_hbm.at[idx_v.at[j]], rows_v.at[b], gsem.at[b]).wait()
            write(j, b)
        for b in range(K):
            nj = (g + 1) * K + b
            @pl.when(nj < NCH)
            def _():
                pltpu.make_async_copy(
                    rows_v.at[b],
                    out_hbm.at[pl.ds(base, CHUNK)], wsem.at[b]).wait()
                gather(nj, b)
        return carry

    lax.fori_loop(0, NG, body, 0)
    for b in range(K):
        pltpu.make_async_copy(
            rows_v.at[b], out_hbm.at[pl.ds(base, CHUNK)], wsem.at[b]).wait()


@jax.jit
def _lookup(idx, table):
    mesh = plsc.VectorSubcoreMesh(core_axis_name="c", subcore_axis_name="s")
    return pl.kernel(
        _emb,
        out_type=jax.ShapeDtypeStruct((N, EMBED), jnp.float32),
        mesh=mesh,
        scratch_types=[
            pltpu.VMEM((NCH, CHUNK), jnp.int32),
            pltpu.VMEM((K, CHUNK, EMBED), jnp.float32),
            pltpu.SemaphoreType.DMA((K,)),
            pltpu.SemaphoreType.DMA((K,)),
        ],
    )(idx, table)


def kernel(x, hidden_h, hidden_c, table):
    idx = x.reshape(NW, NCH, CHUNK)
    embeds = _lookup(idx, table).reshape(BATCH, SEQ, EMBED)
    return (embeds, hidden_h, hidden_c)


# speedup vs baseline: 7.6147x; 1.0102x over previous
import functools

import jax
import jax.numpy as jnp
from jax import lax
from jax.experimental import pallas as pl
from jax.experimental.pallas import tpu as pltpu
from jax.experimental.pallas import tpu_sc as plsc

BATCH = 1024
SEQ = 200
EMBED = 128
N = BATCH * SEQ
NW = 32
PER_W = N // NW
CHUNK = 64
NCH = PER_W // CHUNK
K = 10
NG = NCH // K
HB = K // 2


def _emb(idx_hbm, table_hbm, out_hbm, idx_v, rows_v, gsem, wsem):
    nc = 2
    wid = lax.axis_index("s") * nc + lax.axis_index("c")
    base = wid * PER_W
    pltpu.sync_copy(idx_hbm.at[wid], idx_v)

    def gather(j, b):
        return pltpu.async_copy(
            table_hbm.at[idx_v.at[j]], rows_v.at[pl.ds(b * CHUNK, CHUNK)],
            gsem.at[b])

    def write_block(g, h):
        j0 = g * K + h * HB
        return pltpu.async_copy(
            rows_v.at[pl.ds(h * HB * CHUNK, HB * CHUNK)],
            out_hbm.at[pl.ds(base + j0 * CHUNK, HB * CHUNK)], wsem.at[h])

    def wait_write(h):
        pltpu.make_async_copy(
            rows_v.at[pl.ds(h * HB * CHUNK, HB * CHUNK)],
            out_hbm.at[pl.ds(base, HB * CHUNK)], wsem.at[h]).wait()

    for b in range(K):
        gather(b, b)

    def body(g, carry):
        for h in range(2):
            for b in range(h * HB, h * HB + HB):
                pltpu.make_async_copy(
                    table_hbm.at[idx_v.at[g * K + b]],
                    rows_v.at[pl.ds(b * CHUNK, CHUNK)], gsem.at[b]).wait()
            write_block(g, h)
        for h in range(2):
            @pl.when(g + 1 < NG)
            def _():
                wait_write(h)
                for b in range(h * HB, h * HB + HB):
                    gather((g + 1) * K + b, b)
        return carry

    lax.fori_loop(0, NG, body, 0)
    for h in range(2):
        wait_write(h)


@jax.jit
def _lookup(idx, table):
    mesh = plsc.VectorSubcoreMesh(core_axis_name="c", subcore_axis_name="s")
    return pl.kernel(
        _emb,
        out_type=jax.ShapeDtypeStruct((N, EMBED), jnp.float32),
        mesh=mesh,
        scratch_types=[
            pltpu.VMEM((NCH, CHUNK), jnp.int32),
            pltpu.VMEM((K * CHUNK, EMBED), jnp.float32),
            pltpu.SemaphoreType.DMA((K,)),
            pltpu.SemaphoreType.DMA((2,)),
        ],
    )(idx, table)


def kernel(x, hidden_h, hidden_c, table):
    idx = x.reshape(NW, NCH, CHUNK)
    embeds = _lookup(idx, table).reshape(BATCH, SEQ, EMBED)
    return (embeds, hidden_h, hidden_c)
